# baseline (speedup 1.0000x reference)
import functools

import jax
import jax.numpy as jnp
from jax import lax
from jax.experimental import pallas as pl
from jax.experimental.pallas import tpu as pltpu
from jax.experimental.pallas import tpu_sc as plsc

N = 6890

_info = plsc.get_sparse_core_info()
NC, NS, L = _info.num_cores, _info.num_subcores, _info.num_lanes
NW = NC * NS
CHUNK = 256
PAD = CHUNK * NW
NGATH = CHUNK // 128
NVEC = CHUNK // L
NP2P = 6912


def _sc_body(p2p_hbm, ca_hbm, cb_hbm, flat_hbm, w_hbm, out_hbm,
             p2p_v, ca_v, cb_v, idx_v, vals_v, w_v, acc_v, tmp_v, out_v,
             shared, sem):
    cid = lax.axis_index("c")
    sid = lax.axis_index("s")
    wid = sid * NC + cid
    base = wid * CHUNK

    pltpu.sync_copy(p2p_hbm, p2p_v)
    pltpu.sync_copy(ca_hbm.at[pl.ds(base, CHUNK)], ca_v)
    pltpu.sync_copy(cb_hbm.at[pl.ds(base, CHUNK)], cb_v)
    pltpu.sync_copy(w_hbm.at[pl.ds(base, CHUNK)], w_v)

    for j in range(NVEC):
        ca = ca_v[pl.ds(j * L, L)]
        cb = cb_v[pl.ds(j * L, L)]
        pred = plsc.load_gather(p2p_v, [ca])
        flat = pred * N + cb
        idx_v[j // 8, pl.ds((j % 8) * L, L)] = flat

    copies = [
        pltpu.async_copy(flat_hbm.at[idx_v.at[g]], vals_v.at[g], sem)
        for g in range(NGATH)
    ]
    for cp in copies:
        cp.wait()

    acc = jnp.zeros((L,), jnp.float32)
    for j in range(NVEC):
        v = vals_v[j // 8, pl.ds((j % 8) * L, L)]
        w = w_v[pl.ds(j * L, L)]
        acc = acc + v * w
    acc_v[...] = acc

    pltpu.sync_copy(acc_v, out_hbm.at[wid])


_sc_call = functools.partial(
    pl.kernel,
    mesh=plsc.VectorSubcoreMesh(core_axis_name="c", subcore_axis_name="s"),
    out_type=jax.ShapeDtypeStruct((NW, L), jnp.float32),
    scratch_types=[
        pltpu.VMEM((NP2P,), jnp.int32),
        pltpu.VMEM((CHUNK,), jnp.int32),
        pltpu.VMEM((CHUNK,), jnp.int32),
        pltpu.VMEM((NGATH, 128), jnp.int32),
        pltpu.VMEM((NGATH, 128), jnp.float32),
        pltpu.VMEM((CHUNK,), jnp.float32),
        pltpu.VMEM((L,), jnp.float32),
        pltpu.VMEM((L,), jnp.float32),
        pltpu.VMEM((L,), jnp.float32),
        pltpu.VMEM_SHARED((NS, L), jnp.float32),
        pltpu.SemaphoreType.DMA,
    ],
    compiler_params=pltpu.CompilerParams(needs_layout_passes=False),
)(_sc_body)


def kernel(p2p12, dist_b, corr_a, corr_b):
    p2p = jnp.pad(p2p12.astype(jnp.int32), (0, NP2P - N))
    ca = jnp.pad(corr_a.astype(jnp.int32), (0, PAD - N))
    cb = jnp.pad(corr_b.astype(jnp.int32), (0, PAD - N))
    flat = dist_b.reshape(-1)
    w = (jnp.arange(PAD, dtype=jnp.int32) < N).astype(jnp.float32)
    out = _sc_call(p2p, ca, cb, flat, w)
    return jnp.sum(out) * jnp.float32(1.0 / N)
